# baseline (speedup 1.0000x reference)
import functools
import jax
import jax.numpy as jnp
from jax import lax
from jax.experimental import pallas as pl
from jax.experimental.pallas import tpu as pltpu
from jax.experimental.pallas import tpu_sc as plsc

B, N, M, K = 16, 2048, 512, 64
C_IN = 128
EPS = 1e-5
R2 = 0.2 * 0.2

NROWS = M * B
NW = 32
RPW = NROWS // NW
RB = 16

ROWS = NROWS * K
GPW = ROWS // NW
CH = 128
NCH = GPW // CH

CHUNK = 16384
NSTEP = ROWS // CHUNK


def _fps_step(i, px, py, pz, md, pb0, pb1, pb2, pp):
    iota = lax.broadcasted_iota(jnp.int32, (B, N), 1)
    rowmax = jnp.max(md, axis=1, keepdims=True)
    far = jnp.min(jnp.where(md == rowmax, iota, N), axis=1)

    onehot = (iota == far[:, None]).astype(jnp.float32)
    cx = jnp.sum(px * onehot, axis=1, keepdims=True)
    cy = jnp.sum(py * onehot, axis=1, keepdims=True)
    cz = jnp.sum(pz * onehot, axis=1, keepdims=True)

    dx = px - cx
    dy = py - cy
    dz = pz - cz
    d = dx * dx + dy * dy + dz * dz

    qp = (cx.astype(jnp.bfloat16).astype(jnp.float32) * pb0
          + cy.astype(jnp.bfloat16).astype(jnp.float32) * pb1) \
        + cz.astype(jnp.bfloat16).astype(jnp.float32) * pb2
    qq = (cx * cx + cy * cy) + cz * cz
    d2row = (qq + pp) - 2.0 * qp
    md2 = jnp.minimum(md, d)
    nrow = jnp.concatenate([cx, cy, cz], axis=1)
    return md2, d2row, nrow


def _fps_body(xyz_ref, d2_ref, nxyz_ref, md_ref, pb_ref, pp_ref):
    i = pl.program_id(0)

    px = xyz_ref[:, 0, :]
    py = xyz_ref[:, 1, :]
    pz = xyz_ref[:, 2, :]

    @pl.when(i == 0)
    def _():
        md_ref[...] = jnp.full((B, N), 1e10, dtype=jnp.float32)
        pb_ref[0] = px.astype(jnp.bfloat16).astype(jnp.float32)
        pb_ref[1] = py.astype(jnp.bfloat16).astype(jnp.float32)
        pb_ref[2] = pz.astype(jnp.bfloat16).astype(jnp.float32)
        pp_ref[...] = (px * px + py * py) + pz * pz

    pb0, pb1, pb2 = pb_ref[0], pb_ref[1], pb_ref[2]
    pp = pp_ref[...]
    md = md_ref[...]
    md, d2a, nxa = _fps_step(i, px, py, pz, md, pb0, pb1, pb2, pp)
    md, d2b, nxb = _fps_step(i, px, py, pz, md, pb0, pb1, pb2, pp)
    md_ref[...] = md
    d2_ref[0] = d2a
    d2_ref[1] = d2b
    nxyz_ref[0] = nxa
    nxyz_ref[1] = nxb


def _fps_d2(xyz):
    return pl.pallas_call(
        _fps_body,
        grid=(M // 2,),
        in_specs=[pl.BlockSpec((B, 3, N), lambda i: (0, 0, 0))],
        out_specs=[
            pl.BlockSpec((2, B, N), lambda i: (i, 0, 0)),
            pl.BlockSpec((2, B, 3), lambda i: (i, 0, 0)),
        ],
        out_shape=[
            jax.ShapeDtypeStruct((M, B, N), jnp.float32),
            jax.ShapeDtypeStruct((M, B, 3), jnp.float32),
        ],
        scratch_shapes=[
            pltpu.VMEM((B, N), jnp.float32),
            pltpu.VMEM((3, B, N), jnp.float32),
            pltpu.VMEM((B, N), jnp.float32),
        ],
    )(xyz)


def _compact_body(d2_hbm, out_hbm, rows_v, bufa_v, bufb_v, out_v):
    wid = lax.axis_index("s") * 2 + lax.axis_index("c")
    r0 = wid * RPW

    def blk_body(jb, _):
        pltpu.sync_copy(d2_hbm.at[pl.ds(r0 + jb * RB, RB)], rows_v)

        def pair_body(jp, _):
            ja = 2 * jp
            jb2 = 2 * jp + 1
            ra = r0 + jb * RB + ja
            rb = ra + 1
            base_a = (ra % B) * N
            base_b = (rb % B) * N

            def chunk_body(c, carry):
                cnt_a, cnt_b = carry
                va = rows_v[ja, pl.ds(c * 16, 16)]
                vb = rows_v[jb2, pl.ds(c * 16, 16)]
                mask_a = va < R2
                mask_b = vb < R2
                iot = lax.iota(jnp.int32, 16) + c * 16
                csum_a = plsc.cumsum(mask_a.astype(jnp.int32))
                csum_b = plsc.cumsum(mask_b.astype(jnp.int32))
                plsc.store_scatter(bufa_v, [cnt_a + csum_a - 1],
                                   iot + base_a, mask=mask_a)
                plsc.store_scatter(bufb_v, [cnt_b + csum_b - 1],
                                   iot + base_b, mask=mask_b)
                return (cnt_a + csum_a[15], cnt_b + csum_b[15])

            cnt_a, cnt_b = lax.fori_loop(
                0, N // 16, chunk_body, (jnp.int32(0), jnp.int32(0)))
            first_a = jnp.where(cnt_a > 0, bufa_v[pl.ds(0, 16)][0], base_a)
            first_b = jnp.where(cnt_b > 0, bufb_v[pl.ds(0, 16)][0], base_b)
            for jj in range(K // 16):
                slot = lax.iota(jnp.int32, 16) + jj * 16
                out_v[jb * RB + ja, pl.ds(jj * 16, 16)] = jnp.where(
                    slot < cnt_a, bufa_v[pl.ds(jj * 16, 16)], first_a)
                out_v[jb * RB + jb2, pl.ds(jj * 16, 16)] = jnp.where(
                    slot < cnt_b, bufb_v[pl.ds(jj * 16, 16)], first_b)
            return 0

        lax.fori_loop(0, RB // 2, pair_body, 0)
        return 0

    lax.fori_loop(0, RPW // RB, blk_body, 0)
    pltpu.sync_copy(out_v, out_hbm.at[pl.ds(r0, RPW)])


def _sc_compact(d2_rows):
    f = functools.partial(
        pl.kernel,
        out_type=jax.ShapeDtypeStruct((NROWS, K), jnp.int32),
        mesh=plsc.VectorSubcoreMesh(core_axis_name="c", subcore_axis_name="s"),
        scratch_types=[
            pltpu.VMEM((RB, N), jnp.float32),
            pltpu.VMEM((N + 16,), jnp.int32),
            pltpu.VMEM((N + 16,), jnp.int32),
            pltpu.VMEM((RPW, K), jnp.int32),
        ],
        compiler_params=pltpu.CompilerParams(needs_layout_passes=False),
    )(_compact_body)
    return f(d2_rows)


GK = 8


def _gather_body(table_hbm, idx_hbm, out_hbm, idx_v, rows_v, sem):
    wid = lax.axis_index("s") * 2 + lax.axis_index("c")
    g0 = wid * GPW

    def body(cc, _):
        base = g0 + cc * (GK * CH)
        pltpu.sync_copy(
            idx_hbm.at[cc + wid * (NCH // GK)], idx_v)
        cps = [
            pltpu.async_copy(table_hbm.at[idx_v.at[j]],
                             rows_v.at[pl.ds(j * CH, CH)], sem)
            for j in range(GK)
        ]
        for cp in cps:
            cp.wait()
        pltpu.sync_copy(rows_v, out_hbm.at[pl.ds(base, GK * CH)])
        return 0

    lax.fori_loop(0, NCH // GK, body, 0)


def _sc_gather(table, gidx_flat):
    idx3 = gidx_flat.reshape(NW * (NCH // GK), GK, CH)
    f = functools.partial(
        pl.kernel,
        out_type=jax.ShapeDtypeStruct((ROWS, 64), jnp.float32),
        mesh=plsc.VectorSubcoreMesh(core_axis_name="c", subcore_axis_name="s"),
        scratch_types=[
            pltpu.VMEM((GK, CH), jnp.int32),
            pltpu.VMEM((GK * CH, 64), jnp.float32),
            pltpu.SemaphoreType.DMA,
        ],
        compiler_params=pltpu.CompilerParams(
            needs_layout_passes=False, use_tc_tiling_on_sc=False),
    )(_gather_body)
    return f(table, idx3)


def _pre1_body(xyz_ref, feat_ref, nxyz_ref, W1_ref, b1_ref, pre1_ref, cc_ref):
    W1 = W1_ref[...]
    w_xyz = W1[:, :3]
    w_f = W1[:, 3:]
    pre = (jnp.dot(w_xyz, xyz_ref[0], precision=lax.Precision.DEFAULT,
                   preferred_element_type=jnp.float32)
           + jnp.dot(w_f, feat_ref[0], precision=lax.Precision.DEFAULT,
                     preferred_element_type=jnp.float32))
    pre1_ref[0] = pre
    nx = nxyz_ref[0]
    cc_ref[0] = b1_ref[...][None, :] - jnp.dot(
        nx, w_xyz.T, precision=lax.Precision.DEFAULT,
        preferred_element_type=jnp.float32)


def _pre1_cc(xyz, feature, nxyz_bm, W1, b1):
    return pl.pallas_call(
        _pre1_body,
        grid=(B,),
        in_specs=[
            pl.BlockSpec((1, 3, N), lambda b: (b, 0, 0)),
            pl.BlockSpec((1, C_IN, N), lambda b: (b, 0, 0)),
            pl.BlockSpec((1, M, 3), lambda b: (b, 0, 0)),
            pl.BlockSpec((128, C_IN + 3), lambda b: (0, 0)),
            pl.BlockSpec((128,), lambda b: (0,)),
        ],
        out_specs=[
            pl.BlockSpec((1, 128, N), lambda b: (b, 0, 0)),
            pl.BlockSpec((1, M, 128), lambda b: (b, 0, 0)),
        ],
        out_shape=[
            jax.ShapeDtypeStruct((B, 128, N), jnp.float32),
            jax.ShapeDtypeStruct((B, M, 128), jnp.float32),
        ],
    )(xyz, feature, nxyz_bm, W1, b1)


DOT = dict(preferred_element_type=jnp.float32)


def _unpack(g_ref):
    u = lax.bitcast_convert_type(g_ref[...], jnp.uint32)
    flo = lax.bitcast_convert_type(u << 16, jnp.float32)
    fhi = lax.bitcast_convert_type(u & jnp.uint32(0xFFFF0000), jnp.float32)
    return jnp.concatenate([flo, fhi], axis=1)


def _p1_body(g_ref, cc_ref, s1_ref, s2_ref):
    i = pl.program_id(0)
    g = _unpack(g_ref).reshape(CHUNK // K, K, 128)
    y = g + cc_ref[...][:, None, :]
    s1 = jnp.sum(y, axis=(0, 1))[None, :]
    s2 = jnp.sum(y * y, axis=(0, 1))[None, :]

    @pl.when(i == 0)
    def _():
        s1_ref[...] = jnp.zeros_like(s1_ref)
        s2_ref[...] = jnp.zeros_like(s2_ref)

    s1_ref[...] += s1
    s2_ref[...] += s2


def _p1(G, cc_rows):
    return pl.pallas_call(
        _p1_body,
        grid=(NSTEP,),
        in_specs=[
            pl.BlockSpec((CHUNK, 64), lambda i: (i, 0)),
            pl.BlockSpec((CHUNK // K, 128), lambda i: (i, 0)),
        ],
        out_specs=[pl.BlockSpec((1, 128), lambda i: (0, 0))] * 2,
        out_shape=[jax.ShapeDtypeStruct((1, 128), jnp.float32)] * 2,
        )(G, cc_rows)


def _p2_body(g_ref, ccz_ref, a1_ref, v_ref, mom_ref):
    i = pl.program_id(0)
    g = _unpack(g_ref).reshape(CHUNK // K, K, 128)
    z1 = jnp.maximum(g * a1_ref[...][None, :, :] + ccz_ref[...][:, None, :],
                     0.0).reshape(CHUNK, 128)
    v = jnp.sum(z1, axis=0)[None, :]
    mom = lax.dot_general(z1, z1, (((0,), (0,)), ((), ())), **DOT)

    @pl.when(i == 0)
    def _():
        v_ref[...] = jnp.zeros_like(v_ref)
        mom_ref[...] = jnp.zeros_like(mom_ref)

    v_ref[...] += v
    mom_ref[...] += mom


def _p2(G, ccz, a1):
    return pl.pallas_call(
        _p2_body,
        grid=(NSTEP,),
        in_specs=[
            pl.BlockSpec((CHUNK, 64), lambda i: (i, 0)),
            pl.BlockSpec((CHUNK // K, 128), lambda i: (i, 0)),
            pl.BlockSpec((1, 128), lambda i: (0, 0)),
        ],
        out_specs=[
            pl.BlockSpec((1, 128), lambda i: (0, 0)),
            pl.BlockSpec((128, 128), lambda i: (0, 0)),
        ],
        out_shape=[
            jax.ShapeDtypeStruct((1, 128), jnp.float32),
            jax.ShapeDtypeStruct((128, 128), jnp.float32),
        ],
        )(G, ccz, a1)


def _asm_body(v_ref, mom_ref, wt_ref, b_ref, s1_ref, s2_ref):
    wt = wt_ref[...]
    u = jnp.dot(v_ref[...], wt, **DOT)
    t = jnp.dot(mom_ref[...], wt, **DOT)
    q = jnp.sum(t * wt, axis=0)[None, :]
    b = b_ref[...]
    s1_ref[...] = u + float(ROWS) * b
    s2_ref[...] = q + 2.0 * b * u + float(ROWS) * b * b


def _asm_stats(v, mom, WT, b):
    cout = WT.shape[1]
    return pl.pallas_call(
        _asm_body,
        in_specs=[
            pl.BlockSpec(v.shape, lambda: (0, 0)),
            pl.BlockSpec(mom.shape, lambda: (0, 0)),
            pl.BlockSpec(WT.shape, lambda: (0, 0)),
            pl.BlockSpec((1, cout), lambda: (0, 0)),
        ],
        out_specs=[pl.BlockSpec((1, cout), lambda: (0, 0))] * 2,
        out_shape=[jax.ShapeDtypeStruct((1, cout), jnp.float32)] * 2,
        )(v, mom, WT, b.reshape(1, -1))


def _p3_body(g_ref, ccz_ref, a1_ref, w2p_ref, e2_ref, v_ref, mom_ref):
    i = pl.program_id(0)
    g = _unpack(g_ref).reshape(CHUNK // K, K, 128)
    z1 = jnp.maximum(g * a1_ref[...][None, :, :] + ccz_ref[...][:, None, :],
                     0.0).reshape(CHUNK, 128)
    z2 = jnp.maximum(jnp.dot(z1, w2p_ref[...], **DOT) + e2_ref[...], 0.0)
    v = jnp.sum(z2, axis=0)[None, :]
    mom = lax.dot_general(z2, z2, (((0,), (0,)), ((), ())), **DOT)

    @pl.when(i == 0)
    def _():
        v_ref[...] = jnp.zeros_like(v_ref)
        mom_ref[...] = jnp.zeros_like(mom_ref)

    v_ref[...] += v
    mom_ref[...] += mom


def _p3(G, ccz, a1, W2p, e2):
    return pl.pallas_call(
        _p3_body,
        grid=(NSTEP,),
        in_specs=[
            pl.BlockSpec((CHUNK, 64), lambda i: (i, 0)),
            pl.BlockSpec((CHUNK // K, 128), lambda i: (i, 0)),
            pl.BlockSpec((1, 128), lambda i: (0, 0)),
            pl.BlockSpec((128, 128), lambda i: (0, 0)),
            pl.BlockSpec((1, 128), lambda i: (0, 0)),
        ],
        out_specs=[
            pl.BlockSpec((1, 128), lambda i: (0, 0)),
            pl.BlockSpec((128, 128), lambda i: (0, 0)),
        ],
        out_shape=[
            jax.ShapeDtypeStruct((1, 128), jnp.float32),
            jax.ShapeDtypeStruct((128, 128), jnp.float32),
        ],
        )(G, ccz, a1, W2p, e2)


def _p4_body(g_ref, ccz_ref, a1_ref, w2p_ref, e2_ref, w3p_ref, e3_ref, out_ref):
    g = _unpack(g_ref).reshape(CHUNK // K, K, 128)
    z1 = jnp.maximum(g * a1_ref[...][None, :, :] + ccz_ref[...][:, None, :],
                     0.0).reshape(CHUNK, 128)
    z2 = jnp.maximum(jnp.dot(z1, w2p_ref[...], **DOT) + e2_ref[...], 0.0)
    z3 = jnp.maximum(jnp.dot(z2, w3p_ref[...], **DOT) + e3_ref[...], 0.0)
    out_ref[...] = jnp.max(z3.reshape(CHUNK // K, K, 256), axis=1)


def _p4(G, ccz, a1, W2p, e2, W3p, e3):
    return pl.pallas_call(
        _p4_body,
        grid=(NSTEP,),
        in_specs=[
            pl.BlockSpec((CHUNK, 64), lambda i: (i, 0)),
            pl.BlockSpec((CHUNK // K, 128), lambda i: (i, 0)),
            pl.BlockSpec((1, 128), lambda i: (0, 0)),
            pl.BlockSpec((128, 128), lambda i: (0, 0)),
            pl.BlockSpec((1, 128), lambda i: (0, 0)),
            pl.BlockSpec((128, 256), lambda i: (0, 0)),
            pl.BlockSpec((1, 256), lambda i: (0, 0)),
        ],
        out_specs=pl.BlockSpec((CHUNK // K, 256), lambda i: (i, 0)),
        out_shape=jax.ShapeDtypeStruct((NROWS, 256), jnp.float32),
        )(G, ccz, a1, W2p, e2, W3p, e3)


def _affine(g, be, s1, s2):
    n = float(ROWS)
    mean = s1[0] / n
    var = s2[0] / n - mean * mean
    a = g / jnp.sqrt(var + EPS)
    c = be - mean * a
    return a, c


def kernel(xyz, feature, W1, b1, g1, be1, W2, b2, g2, be2, W3, b3, g3, be3):
    d2, nxyz = _fps_d2(xyz)
    new_xyz = jnp.transpose(nxyz, (1, 2, 0))
    gidx = _sc_compact(d2.reshape(NROWS, N))
    pre1, cc = _pre1_cc(xyz, feature, jnp.transpose(nxyz, (1, 0, 2)), W1, b1)
    pre_bf = jnp.transpose(pre1.astype(jnp.bfloat16), (0, 2, 1))
    plo = lax.bitcast_convert_type(pre_bf[:, :, :64], jnp.uint16).astype(jnp.uint32)
    phi = lax.bitcast_convert_type(pre_bf[:, :, 64:], jnp.uint16).astype(jnp.uint32)
    packed = lax.bitcast_convert_type(plo | (phi << 16), jnp.float32)
    G = _sc_gather(packed.reshape(B * N, 64), gidx.reshape(-1))
    cc_rows = jnp.transpose(cc, (1, 0, 2)).reshape(NROWS, 128)
    s1, s2 = _p1(G, cc_rows)
    a1, c1 = _affine(g1, be1, s1, s2)
    a1r = a1[None, :]
    ccz = cc_rows * a1r + c1[None, :]
    v1, mom1 = _p2(G, ccz, a1r)
    t1, t2 = _asm_stats(v1, mom1, W2.T, b2)
    a2, c2 = _affine(g2, be2, t1, t2)
    W2p = W2.T * a2[None, :]
    e2 = (a2 * b2 + c2)[None, :]
    v2, mom2 = _p3(G, ccz, a1r, W2p, e2)
    u1, u2 = _asm_stats(v2, mom2, W3.T, b3)
    a3, c3 = _affine(g3, be3, u1, u2)
    W3p = W3.T * a3[None, :]
    e3 = (a3 * b3 + c3)[None, :]
    pooled = _p4(G, ccz, a1r, W2p, e2, W3p, e3)
    nf = jnp.transpose(pooled.reshape(M, B, 256), (1, 2, 0))
    return (new_xyz, nf)


# speedup vs baseline: 1.0041x; 1.0041x over previous
import functools
import jax
import jax.numpy as jnp
from jax import lax
from jax.experimental import pallas as pl
from jax.experimental.pallas import tpu as pltpu
from jax.experimental.pallas import tpu_sc as plsc

B, N, M, K = 16, 2048, 512, 64
C_IN = 128
EPS = 1e-5
R2 = 0.2 * 0.2

NROWS = M * B
NW = 32
RPW = NROWS // NW
RB = 16

ROWS = NROWS * K
GPW = ROWS // NW
CH = 128
NCH = GPW // CH

CHUNK = 16384
NSTEP = ROWS // CHUNK


def _fps_body(xyz_ref, d2_ref, nxyz_ref, md_ref, pb_ref, pp_ref):
    i = pl.program_id(0)

    px = xyz_ref[:, 0, :]
    py = xyz_ref[:, 1, :]
    pz = xyz_ref[:, 2, :]

    @pl.when(i == 0)
    def _():
        md_ref[...] = jnp.full((B, N), 1e10, dtype=jnp.float32)
        pb_ref[0] = px.astype(jnp.bfloat16).astype(jnp.float32)
        pb_ref[1] = py.astype(jnp.bfloat16).astype(jnp.float32)
        pb_ref[2] = pz.astype(jnp.bfloat16).astype(jnp.float32)
        pp_ref[...] = (px * px + py * py) + pz * pz

    md = md_ref[...]
    iota = lax.broadcasted_iota(jnp.int32, (B, N), 1)
    rowmax = jnp.max(md, axis=1, keepdims=True)
    far = jnp.min(jnp.where(md == rowmax, iota, N), axis=1)

    onehot = (iota == far[:, None]).astype(jnp.float32)
    cx = jnp.sum(px * onehot, axis=1, keepdims=True)
    cy = jnp.sum(py * onehot, axis=1, keepdims=True)
    cz = jnp.sum(pz * onehot, axis=1, keepdims=True)

    dx = px - cx
    dy = py - cy
    dz = pz - cz
    d = dx * dx + dy * dy + dz * dz

    qp = (cx.astype(jnp.bfloat16).astype(jnp.float32) * pb_ref[0]
          + cy.astype(jnp.bfloat16).astype(jnp.float32) * pb_ref[1]) \
        + cz.astype(jnp.bfloat16).astype(jnp.float32) * pb_ref[2]
    qq = (cx * cx + cy * cy) + cz * cz
    d2_ref[0] = (qq + pp_ref[...]) - 2.0 * qp

    md_ref[...] = jnp.minimum(md, d)
    nxyz_ref[0] = jnp.concatenate([cx, cy, cz], axis=1)


def _fps_d2(xyz):
    return pl.pallas_call(
        _fps_body,
        grid=(M,),
        in_specs=[pl.BlockSpec((B, 3, N), lambda i: (0, 0, 0))],
        out_specs=[
            pl.BlockSpec((1, B, N), lambda i: (i, 0, 0)),
            pl.BlockSpec((1, B, 3), lambda i: (i, 0, 0)),
        ],
        out_shape=[
            jax.ShapeDtypeStruct((M, B, N), jnp.float32),
            jax.ShapeDtypeStruct((M, B, 3), jnp.float32),
        ],
        scratch_shapes=[
            pltpu.VMEM((B, N), jnp.float32),
            pltpu.VMEM((3, B, N), jnp.float32),
            pltpu.VMEM((B, N), jnp.float32),
        ],
    )(xyz)


def _compact_body(d2_hbm, out_hbm, rows_v, bufa_v, bufb_v, out_v):
    wid = lax.axis_index("s") * 2 + lax.axis_index("c")
    r0 = wid * RPW

    def blk_body(jb, _):
        pltpu.sync_copy(d2_hbm.at[pl.ds(r0 + jb * RB, RB)], rows_v)

        def pair_body(jp, _):
            ja = 2 * jp
            jb2 = 2 * jp + 1
            ra = r0 + jb * RB + ja
            rb = ra + 1
            base_a = (ra % B) * N
            base_b = (rb % B) * N

            def chunk_body(c, carry):
                cnt_a, cnt_b = carry
                va = rows_v[ja, pl.ds(c * 16, 16)]
                vb = rows_v[jb2, pl.ds(c * 16, 16)]
                mask_a = va < R2
                mask_b = vb < R2
                iot = lax.iota(jnp.int32, 16) + c * 16
                csum_a = plsc.cumsum(mask_a.astype(jnp.int32))
                csum_b = plsc.cumsum(mask_b.astype(jnp.int32))
                plsc.store_scatter(bufa_v, [cnt_a + csum_a - 1],
                                   iot + base_a, mask=mask_a)
                plsc.store_scatter(bufb_v, [cnt_b + csum_b - 1],
                                   iot + base_b, mask=mask_b)
                return (cnt_a + csum_a[15], cnt_b + csum_b[15])

            cnt_a, cnt_b = lax.fori_loop(
                0, N // 16, chunk_body, (jnp.int32(0), jnp.int32(0)))
            first_a = jnp.where(cnt_a > 0, bufa_v[pl.ds(0, 16)][0], base_a)
            first_b = jnp.where(cnt_b > 0, bufb_v[pl.ds(0, 16)][0], base_b)
            for jj in range(K // 16):
                slot = lax.iota(jnp.int32, 16) + jj * 16
                out_v[jb * RB + ja, pl.ds(jj * 16, 16)] = jnp.where(
                    slot < cnt_a, bufa_v[pl.ds(jj * 16, 16)], first_a)
                out_v[jb * RB + jb2, pl.ds(jj * 16, 16)] = jnp.where(
                    slot < cnt_b, bufb_v[pl.ds(jj * 16, 16)], first_b)
            return 0

        lax.fori_loop(0, RB // 2, pair_body, 0)
        return 0

    lax.fori_loop(0, RPW // RB, blk_body, 0)
    pltpu.sync_copy(out_v, out_hbm.at[pl.ds(r0, RPW)])


def _sc_compact(d2_rows):
    f = functools.partial(
        pl.kernel,
        out_type=jax.ShapeDtypeStruct((NROWS, K), jnp.int32),
        mesh=plsc.VectorSubcoreMesh(core_axis_name="c", subcore_axis_name="s"),
        scratch_types=[
            pltpu.VMEM((RB, N), jnp.float32),
            pltpu.VMEM((N + 16,), jnp.int32),
            pltpu.VMEM((N + 16,), jnp.int32),
            pltpu.VMEM((RPW, K), jnp.int32),
        ],
        compiler_params=pltpu.CompilerParams(needs_layout_passes=False),
    )(_compact_body)
    return f(d2_rows)


GK = 8


def _gather_body(table_hbm, idx_hbm, out_hbm, idx_v, rows_v, sem):
    wid = lax.axis_index("s") * 2 + lax.axis_index("c")
    g0 = wid * GPW

    def body(cc, _):
        base = g0 + cc * (GK * CH)
        pltpu.sync_copy(
            idx_hbm.at[cc + wid * (NCH // GK)], idx_v)
        cps = [
            pltpu.async_copy(table_hbm.at[idx_v.at[j]],
                             rows_v.at[pl.ds(j * CH, CH)], sem)
            for j in range(GK)
        ]
        for cp in cps:
            cp.wait()
        pltpu.sync_copy(rows_v, out_hbm.at[pl.ds(base, GK * CH)])
        return 0

    lax.fori_loop(0, NCH // GK, body, 0)


def _sc_gather(table, gidx_flat):
    idx3 = gidx_flat.reshape(NW * (NCH // GK), GK, CH)
    f = functools.partial(
        pl.kernel,
        out_type=jax.ShapeDtypeStruct((ROWS, 64), jnp.float32),
        mesh=plsc.VectorSubcoreMesh(core_axis_name="c", subcore_axis_name="s"),
        scratch_types=[
            pltpu.VMEM((GK, CH), jnp.int32),
            pltpu.VMEM((GK * CH, 64), jnp.float32),
            pltpu.SemaphoreType.DMA,
        ],
        compiler_params=pltpu.CompilerParams(
            needs_layout_passes=False, use_tc_tiling_on_sc=False),
    )(_gather_body)
    return f(table, idx3)


def _pre1_body(xyz_ref, feat_ref, nxyz_ref, W1_ref, b1_ref, pre1_ref, cc_ref):
    xt = jnp.transpose(xyz_ref[0], (1, 0))
    ft = jnp.transpose(feat_ref[0], (1, 0))
    W1 = W1_ref[...]
    w_xyz = W1[:, :3]
    w_f = W1[:, 3:]
    pre = (jnp.dot(xt, w_xyz.T, precision=lax.Precision.DEFAULT,
                   preferred_element_type=jnp.float32)
           + jnp.dot(ft, w_f.T, precision=lax.Precision.DEFAULT,
                     preferred_element_type=jnp.float32))
    pre1_ref[0] = pre
    nx = nxyz_ref[0]
    cc_ref[0] = b1_ref[...][None, :] - jnp.dot(
        nx, w_xyz.T, precision=lax.Precision.DEFAULT,
        preferred_element_type=jnp.float32)


def _pre1_cc(xyz, feature, nxyz_bm, W1, b1):
    return pl.pallas_call(
        _pre1_body,
        grid=(B,),
        in_specs=[
            pl.BlockSpec((1, 3, N), lambda b: (b, 0, 0)),
            pl.BlockSpec((1, C_IN, N), lambda b: (b, 0, 0)),
            pl.BlockSpec((1, M, 3), lambda b: (b, 0, 0)),
            pl.BlockSpec((128, C_IN + 3), lambda b: (0, 0)),
            pl.BlockSpec((128,), lambda b: (0,)),
        ],
        out_specs=[
            pl.BlockSpec((1, N, 128), lambda b: (b, 0, 0)),
            pl.BlockSpec((1, M, 128), lambda b: (b, 0, 0)),
        ],
        out_shape=[
            jax.ShapeDtypeStruct((B, N, 128), jnp.float32),
            jax.ShapeDtypeStruct((B, M, 128), jnp.float32),
        ],
    )(xyz, feature, nxyz_bm, W1, b1)


DOT = dict(preferred_element_type=jnp.float32)


def _unpack(g_ref):
    u = lax.bitcast_convert_type(g_ref[...], jnp.uint32)
    flo = lax.bitcast_convert_type(u << 16, jnp.float32)
    fhi = lax.bitcast_convert_type(u & jnp.uint32(0xFFFF0000), jnp.float32)
    return jnp.concatenate([flo, fhi], axis=1)


def _p1_body(g_ref, cc_ref, s1_ref, s2_ref):
    i = pl.program_id(0)
    g = _unpack(g_ref).reshape(CHUNK // K, K, 128)
    y = g + cc_ref[...][:, None, :]
    s1 = jnp.sum(y, axis=(0, 1))[None, :]
    s2 = jnp.sum(y * y, axis=(0, 1))[None, :]

    @pl.when(i == 0)
    def _():
        s1_ref[...] = jnp.zeros_like(s1_ref)
        s2_ref[...] = jnp.zeros_like(s2_ref)

    s1_ref[...] += s1
    s2_ref[...] += s2


def _p1(G, cc_rows):
    return pl.pallas_call(
        _p1_body,
        grid=(NSTEP,),
        in_specs=[
            pl.BlockSpec((CHUNK, 64), lambda i: (i, 0)),
            pl.BlockSpec((CHUNK // K, 128), lambda i: (i, 0)),
        ],
        out_specs=[pl.BlockSpec((1, 128), lambda i: (0, 0))] * 2,
        out_shape=[jax.ShapeDtypeStruct((1, 128), jnp.float32)] * 2,
        )(G, cc_rows)


def _p2_body(g_ref, ccz_ref, a1_ref, v_ref, mom_ref):
    i = pl.program_id(0)
    g = _unpack(g_ref).reshape(CHUNK // K, K, 128)
    z1 = jnp.maximum(g * a1_ref[...][None, :, :] + ccz_ref[...][:, None, :],
                     0.0).reshape(CHUNK, 128)
    v = jnp.sum(z1, axis=0)[None, :]
    mom = lax.dot_general(z1, z1, (((0,), (0,)), ((), ())), **DOT)

    @pl.when(i == 0)
    def _():
        v_ref[...] = jnp.zeros_like(v_ref)
        mom_ref[...] = jnp.zeros_like(mom_ref)

    v_ref[...] += v
    mom_ref[...] += mom


def _p2(G, ccz, a1):
    return pl.pallas_call(
        _p2_body,
        grid=(NSTEP,),
        in_specs=[
            pl.BlockSpec((CHUNK, 64), lambda i: (i, 0)),
            pl.BlockSpec((CHUNK // K, 128), lambda i: (i, 0)),
            pl.BlockSpec((1, 128), lambda i: (0, 0)),
        ],
        out_specs=[
            pl.BlockSpec((1, 128), lambda i: (0, 0)),
            pl.BlockSpec((128, 128), lambda i: (0, 0)),
        ],
        out_shape=[
            jax.ShapeDtypeStruct((1, 128), jnp.float32),
            jax.ShapeDtypeStruct((128, 128), jnp.float32),
        ],
        )(G, ccz, a1)


def _asm_body(v_ref, mom_ref, wt_ref, b_ref, s1_ref, s2_ref):
    wt = wt_ref[...]
    u = jnp.dot(v_ref[...], wt, **DOT)
    t = jnp.dot(mom_ref[...], wt, **DOT)
    q = jnp.sum(t * wt, axis=0)[None, :]
    b = b_ref[...]
    s1_ref[...] = u + float(ROWS) * b
    s2_ref[...] = q + 2.0 * b * u + float(ROWS) * b * b


def _asm_stats(v, mom, WT, b):
    cout = WT.shape[1]
    return pl.pallas_call(
        _asm_body,
        in_specs=[
            pl.BlockSpec(v.shape, lambda: (0, 0)),
            pl.BlockSpec(mom.shape, lambda: (0, 0)),
            pl.BlockSpec(WT.shape, lambda: (0, 0)),
            pl.BlockSpec((1, cout), lambda: (0, 0)),
        ],
        out_specs=[pl.BlockSpec((1, cout), lambda: (0, 0))] * 2,
        out_shape=[jax.ShapeDtypeStruct((1, cout), jnp.float32)] * 2,
        )(v, mom, WT, b.reshape(1, -1))


def _p3_body(g_ref, ccz_ref, a1_ref, w2p_ref, e2_ref, v_ref, mom_ref):
    i = pl.program_id(0)
    g = _unpack(g_ref).reshape(CHUNK // K, K, 128)
    z1 = jnp.maximum(g * a1_ref[...][None, :, :] + ccz_ref[...][:, None, :],
                     0.0).reshape(CHUNK, 128)
    z2 = jnp.maximum(jnp.dot(z1, w2p_ref[...], **DOT) + e2_ref[...], 0.0)
    v = jnp.sum(z2, axis=0)[None, :]
    mom = lax.dot_general(z2, z2, (((0,), (0,)), ((), ())), **DOT)

    @pl.when(i == 0)
    def _():
        v_ref[...] = jnp.zeros_like(v_ref)
        mom_ref[...] = jnp.zeros_like(mom_ref)

    v_ref[...] += v
    mom_ref[...] += mom


def _p3(G, ccz, a1, W2p, e2):
    return pl.pallas_call(
        _p3_body,
        grid=(NSTEP,),
        in_specs=[
            pl.BlockSpec((CHUNK, 64), lambda i: (i, 0)),
            pl.BlockSpec((CHUNK // K, 128), lambda i: (i, 0)),
            pl.BlockSpec((1, 128), lambda i: (0, 0)),
            pl.BlockSpec((128, 128), lambda i: (0, 0)),
            pl.BlockSpec((1, 128), lambda i: (0, 0)),
        ],
        out_specs=[
            pl.BlockSpec((1, 128), lambda i: (0, 0)),
            pl.BlockSpec((128, 128), lambda i: (0, 0)),
        ],
        out_shape=[
            jax.ShapeDtypeStruct((1, 128), jnp.float32),
            jax.ShapeDtypeStruct((128, 128), jnp.float32),
        ],
        )(G, ccz, a1, W2p, e2)


def _p4_body(g_ref, ccz_ref, a1_ref, w2p_ref, e2_ref, w3p_ref, e3_ref, out_ref):
    g = _unpack(g_ref).reshape(CHUNK // K, K, 128)
    z1 = jnp.maximum(g * a1_ref[...][None, :, :] + ccz_ref[...][:, None, :],
                     0.0).reshape(CHUNK, 128)
    z2 = jnp.maximum(jnp.dot(z1, w2p_ref[...], **DOT) + e2_ref[...], 0.0)
    z3 = jnp.maximum(jnp.dot(z2, w3p_ref[...], **DOT) + e3_ref[...], 0.0)
    out_ref[...] = jnp.max(z3.reshape(CHUNK // K, K, 256), axis=1)


def _p4(G, ccz, a1, W2p, e2, W3p, e3):
    return pl.pallas_call(
        _p4_body,
        grid=(NSTEP,),
        in_specs=[
            pl.BlockSpec((CHUNK, 64), lambda i: (i, 0)),
            pl.BlockSpec((CHUNK // K, 128), lambda i: (i, 0)),
            pl.BlockSpec((1, 128), lambda i: (0, 0)),
            pl.BlockSpec((128, 128), lambda i: (0, 0)),
            pl.BlockSpec((1, 128), lambda i: (0, 0)),
            pl.BlockSpec((128, 256), lambda i: (0, 0)),
            pl.BlockSpec((1, 256), lambda i: (0, 0)),
        ],
        out_specs=pl.BlockSpec((CHUNK // K, 256), lambda i: (i, 0)),
        out_shape=jax.ShapeDtypeStruct((NROWS, 256), jnp.float32),
        )(G, ccz, a1, W2p, e2, W3p, e3)


def _affine(g, be, s1, s2):
    n = float(ROWS)
    mean = s1[0] / n
    var = s2[0] / n - mean * mean
    a = g / jnp.sqrt(var + EPS)
    c = be - mean * a
    return a, c


def kernel(xyz, feature, W1, b1, g1, be1, W2, b2, g2, be2, W3, b3, g3, be3):
    d2, nxyz = _fps_d2(xyz)
    new_xyz = jnp.transpose(nxyz, (1, 2, 0))
    gidx = _sc_compact(d2.reshape(NROWS, N))
    pre1, cc = _pre1_cc(xyz, feature, jnp.transpose(nxyz, (1, 0, 2)), W1, b1)
    pre_bf = pre1.astype(jnp.bfloat16)
    plo = lax.bitcast_convert_type(pre_bf[:, :, :64], jnp.uint16).astype(jnp.uint32)
    phi = lax.bitcast_convert_type(pre_bf[:, :, 64:], jnp.uint16).astype(jnp.uint32)
    packed = lax.bitcast_convert_type(plo | (phi << 16), jnp.float32)
    G = _sc_gather(packed.reshape(B * N, 64), gidx.reshape(-1))
    cc_rows = jnp.transpose(cc, (1, 0, 2)).reshape(NROWS, 128)
    s1, s2 = _p1(G, cc_rows)
    a1, c1 = _affine(g1, be1, s1, s2)
    a1r = a1[None, :]
    ccz = cc_rows * a1r + c1[None, :]
    v1, mom1 = _p2(G, ccz, a1r)
    t1, t2 = _asm_stats(v1, mom1, W2.T, b2)
    a2, c2 = _affine(g2, be2, t1, t2)
    W2p = W2.T * a2[None, :]
    e2 = (a2 * b2 + c2)[None, :]
    v2, mom2 = _p3(G, ccz, a1r, W2p, e2)
    u1, u2 = _asm_stats(v2, mom2, W3.T, b3)
    a3, c3 = _affine(g3, be3, u1, u2)
    W3p = W3.T * a3[None, :]
    e3 = (a3 * b3 + c3)[None, :]
    pooled = _p4(G, ccz, a1r, W2p, e2, W3p, e3)
    nf = jnp.transpose(pooled.reshape(M, B, 256), (1, 2, 0))
    return (new_xyz, nf)
